# baseline (speedup 1.0000x reference)
import functools

import jax
import jax.numpy as jnp
from jax import lax
from jax.experimental import pallas as pl
from jax.experimental.pallas import tpu as pltpu

N = 10000
E = 320000
D = 128
H = 128
NEF = 144

BE = 4000
BN = 1000

NC = 2
NS = 16
NW = NC * NS
N2 = 10240

PARTS = 2
EH = E // PARTS
EPW = EH // NW
CHUNK = 40
NCH = EPW // CHUNK

CCHUNK = 80
CEPW = E // NW
CNCH = CEPW // CCHUNK


def _silu(v):
    return v * jax.nn.sigmoid(v)


def _edge0_body(ea_ref, w0_ref, b0_ref, w1_ref, b1_ref, out_ref):
    pre = jnp.dot(ea_ref[...], w0_ref[...], preferred_element_type=jnp.float32)
    h = _silu(pre + b0_ref[...])
    m = _silu(jnp.dot(h, w1_ref[...], preferred_element_type=jnp.float32)
              + b1_ref[...])
    out_ref[...] = m


def _edge0(ea, w0, b0, w1, b1, part):
    nb = EH // BE
    return pl.pallas_call(
        _edge0_body,
        grid=(nb,),
        in_specs=[
            pl.BlockSpec((BE, NEF), lambda i: (i + part * nb, 0)),
            pl.BlockSpec((NEF, H), lambda i: (0, 0)),
            pl.BlockSpec((1, H), lambda i: (0, 0)),
            pl.BlockSpec((H, H), lambda i: (0, 0)),
            pl.BlockSpec((1, H), lambda i: (0, 0)),
        ],
        out_specs=pl.BlockSpec((BE, H), lambda i: (i, 0)),
        out_shape=jax.ShapeDtypeStruct((EH, H), jnp.float32),
    )(ea, w0, b0.reshape(1, H), w1, b1.reshape(1, H))


def _edge_body(ea_ref, gs_ref, gr_ref, we_ref, w1_ref, b1_ref, out_ref):
    pre = jnp.dot(ea_ref[...], we_ref[...], preferred_element_type=jnp.float32)
    h = _silu(pre + (gs_ref[...] + gr_ref[...]))
    m = _silu(jnp.dot(h, w1_ref[...], preferred_element_type=jnp.float32)
              + b1_ref[...])
    out_ref[...] = m


def _edge(ea, gs, gr, we, w1, b1):
    nb = EH // BE
    return pl.pallas_call(
        _edge_body,
        grid=(nb,),
        in_specs=[
            pl.BlockSpec((BE, H), lambda i: (i, 0)),
            pl.BlockSpec((BE, H), lambda i: (i, 0)),
            pl.BlockSpec((BE, H), lambda i: (i, 0)),
            pl.BlockSpec((H, H), lambda i: (0, 0)),
            pl.BlockSpec((H, H), lambda i: (0, 0)),
            pl.BlockSpec((1, H), lambda i: (0, 0)),
        ],
        out_specs=pl.BlockSpec((BE, H), lambda i: (i, 0)),
        out_shape=jax.ShapeDtypeStruct((EH, H), jnp.float32),
    )(ea, gs, gr, we, w1, b1.reshape(1, H))


def _node0_body(x_ref, p00_ref, p01_ref, p10_ref, p11_ref, c0_ref, c1_ref,
                u0w_ref, u0b_ref, u1w_ref, u1b_ref,
                wsn_ref, wrn_ref, b0n_ref,
                x_out, xs_out, xr_out, inv_out):
    s = (p00_ref[...] + p01_ref[...]) + (p10_ref[...] + p11_ref[...])
    c = c0_ref[:, :1] + c1_ref[:, :1]
    inv = 1.0 / jnp.maximum(c, 1.0)
    x1 = x_ref[...] + s * inv
    u = _silu(jnp.dot(x1, u0w_ref[...], preferred_element_type=jnp.float32)
              + u0b_ref[...])
    x2 = x1 + jnp.dot(u, u1w_ref[...], preferred_element_type=jnp.float32) \
        + u1b_ref[...]
    x_out[...] = x2
    xs_out[...] = jnp.dot(x2, wsn_ref[...],
                          preferred_element_type=jnp.float32) + b0n_ref[...]
    xr_out[...] = jnp.dot(x2, wrn_ref[...],
                          preferred_element_type=jnp.float32)
    inv_out[...] = inv


def _node0(x, ps, c0, c1, u0w, u0b, u1w, u1b, wsn, wrn, b0n):
    pspec = pl.BlockSpec((BN, H), lambda i: (i, 0))
    return pl.pallas_call(
        _node0_body,
        grid=(N // BN,),
        in_specs=[pspec] * 7 + [
            pl.BlockSpec((H, 2 * H), lambda i: (0, 0)),
            pl.BlockSpec((1, 2 * H), lambda i: (0, 0)),
            pl.BlockSpec((2 * H, H), lambda i: (0, 0)),
            pl.BlockSpec((1, H), lambda i: (0, 0)),
            pl.BlockSpec((H, H), lambda i: (0, 0)),
            pl.BlockSpec((H, H), lambda i: (0, 0)),
            pl.BlockSpec((1, H), lambda i: (0, 0)),
        ],
        out_specs=[
            pl.BlockSpec((BN, H), lambda i: (i, 0)),
            pl.BlockSpec((BN, H), lambda i: (i, 0)),
            pl.BlockSpec((BN, H), lambda i: (i, 0)),
            pl.BlockSpec((BN, 1), lambda i: (i, 0)),
        ],
        out_shape=[
            jax.ShapeDtypeStruct((N, H), jnp.float32),
            jax.ShapeDtypeStruct((N, H), jnp.float32),
            jax.ShapeDtypeStruct((N, H), jnp.float32),
            jax.ShapeDtypeStruct((N, 1), jnp.float32),
        ],
    )(x, ps[0], ps[1], ps[2], ps[3], c0, c1,
      u0w, u0b.reshape(1, 2 * H), u1w, u1b.reshape(1, H),
      wsn, wrn, b0n.reshape(1, H))


def _node_body(x_ref, p00_ref, p01_ref, p10_ref, p11_ref, inv_ref,
               u0w_ref, u0b_ref, u1w_ref, u1b_ref,
               wsn_ref, wrn_ref, b0n_ref,
               x_out, xs_out, xr_out):
    s = (p00_ref[...] + p01_ref[...]) + (p10_ref[...] + p11_ref[...])
    x1 = x_ref[...] + s * inv_ref[...]
    u = _silu(jnp.dot(x1, u0w_ref[...], preferred_element_type=jnp.float32)
              + u0b_ref[...])
    x2 = x1 + jnp.dot(u, u1w_ref[...], preferred_element_type=jnp.float32) \
        + u1b_ref[...]
    x_out[...] = x2
    xs_out[...] = jnp.dot(x2, wsn_ref[...],
                          preferred_element_type=jnp.float32) + b0n_ref[...]
    xr_out[...] = jnp.dot(x2, wrn_ref[...],
                          preferred_element_type=jnp.float32)


def _node(x, ps, inv, u0w, u0b, u1w, u1b, wsn, wrn, b0n):
    pspec = pl.BlockSpec((BN, H), lambda i: (i, 0))
    return pl.pallas_call(
        _node_body,
        grid=(N // BN,),
        in_specs=[pspec] * 5 + [
            pl.BlockSpec((BN, 1), lambda i: (i, 0)),
            pl.BlockSpec((H, 2 * H), lambda i: (0, 0)),
            pl.BlockSpec((1, 2 * H), lambda i: (0, 0)),
            pl.BlockSpec((2 * H, H), lambda i: (0, 0)),
            pl.BlockSpec((1, H), lambda i: (0, 0)),
            pl.BlockSpec((H, H), lambda i: (0, 0)),
            pl.BlockSpec((H, H), lambda i: (0, 0)),
            pl.BlockSpec((1, H), lambda i: (0, 0)),
        ],
        out_specs=[
            pl.BlockSpec((BN, H), lambda i: (i, 0)),
            pl.BlockSpec((BN, H), lambda i: (i, 0)),
            pl.BlockSpec((BN, H), lambda i: (i, 0)),
        ],
        out_shape=[
            jax.ShapeDtypeStruct((N, H), jnp.float32),
            jax.ShapeDtypeStruct((N, H), jnp.float32),
            jax.ShapeDtypeStruct((N, H), jnp.float32),
        ],
    )(x, ps[0], ps[1], ps[2], ps[3], inv,
      u0w, u0b.reshape(1, 2 * H), u1w, u1b.reshape(1, H),
      wsn, wrn, b0n.reshape(1, H))


def _node_final_body(x_ref, p00_ref, p01_ref, p10_ref, p11_ref, inv_ref,
                     u0w_ref, u0b_ref, u1w_ref, u1b_ref,
                     o0w_ref, o0b_ref, o1w_ref, o1b_ref,
                     o2w_ref, o2b_ref, out_ref):
    s = (p00_ref[...] + p01_ref[...]) + (p10_ref[...] + p11_ref[...])
    x1 = x_ref[...] + s * inv_ref[...]
    u = _silu(jnp.dot(x1, u0w_ref[...], preferred_element_type=jnp.float32)
              + u0b_ref[...])
    x2 = x1 + jnp.dot(u, u1w_ref[...], preferred_element_type=jnp.float32) \
        + u1b_ref[...]
    h = _silu(jnp.dot(x2, o0w_ref[...], preferred_element_type=jnp.float32)
              + o0b_ref[...])
    h = _silu(jnp.dot(h, o1w_ref[...], preferred_element_type=jnp.float32)
              + o1b_ref[...])
    out_ref[...] = jnp.dot(h, o2w_ref[...], preferred_element_type=jnp.float32) \
        + o2b_ref[...]


def _node_final(x, ps, inv, u0w, u0b, u1w, u1b,
                o0w, o0b, o1w, o1b, o2w_pad, o2b_pad):
    pspec = pl.BlockSpec((BN, H), lambda i: (i, 0))
    wspec = pl.BlockSpec((H, H), lambda i: (0, 0))
    bspec = pl.BlockSpec((1, H), lambda i: (0, 0))
    return pl.pallas_call(
        _node_final_body,
        grid=(N // BN,),
        in_specs=[pspec] * 5 + [
            pl.BlockSpec((BN, 1), lambda i: (i, 0)),
            pl.BlockSpec((H, 2 * H), lambda i: (0, 0)),
            pl.BlockSpec((1, 2 * H), lambda i: (0, 0)),
            pl.BlockSpec((2 * H, H), lambda i: (0, 0)),
            bspec, wspec, bspec, wspec, bspec, wspec, bspec,
        ],
        out_specs=pl.BlockSpec((BN, H), lambda i: (i, 0)),
        out_shape=jax.ShapeDtypeStruct((N, H), jnp.float32),
    )(x, ps[0], ps[1], ps[2], ps[3], inv,
      u0w, u0b.reshape(1, 2 * H), u1w, u1b.reshape(1, H),
      o0w, o0b.reshape(1, H), o1w, o1b.reshape(1, H), o2w_pad,
      o2b_pad.reshape(1, H))


def _sc_mesh():
    import jax.experimental.pallas.tpu_sc as plsc
    return plsc.VectorSubcoreMesh(core_axis_name="c", subcore_axis_name="s")


def _sc_gather_add(xs, xr, send, recv, part):
    def body(xs_hbm, xr_hbm, send_hbm, recv_hbm, gs_hbm, gr_hbm,
             sidx, ridx, rows_s0, rows_r0, rows_s1, rows_r1,
             g_s0, g_r0, g_s1, g_r1, t_s0, t_r0, t_s1, t_r1):
        wid = lax.axis_index("c") * NS + lax.axis_index("s")
        ebase = part * EH + wid * EPW
        obase = wid * EPW
        pltpu.sync_copy(send_hbm.at[pl.ds(ebase, EPW)], sidx)
        pltpu.sync_copy(recv_hbm.at[pl.ds(ebase, EPW)], ridx)

        def issue_g(ch, rs, rr, ss, sr):
            pltpu.async_copy(xs_hbm.at[sidx.at[pl.ds(ch * CHUNK, CHUNK)]],
                             rs, ss)
            pltpu.async_copy(xr_hbm.at[ridx.at[pl.ds(ch * CHUNK, CHUNK)]],
                             rr, sr)

        def wait_g(ch, rs, rr, ss, sr):
            pltpu.make_async_copy(
                xs_hbm.at[sidx.at[pl.ds(ch * CHUNK, CHUNK)]], rs, ss).wait()
            pltpu.make_async_copy(
                xr_hbm.at[ridx.at[pl.ds(ch * CHUNK, CHUNK)]], rr, sr).wait()

        def issue_s(ch, rs, rr, ss, sr):
            dst = pl.ds(obase + ch * CHUNK, CHUNK)
            pltpu.async_copy(rs, gs_hbm.at[dst], ss)
            pltpu.async_copy(rr, gr_hbm.at[dst], sr)

        def wait_s(ch, rs, rr, ss, sr):
            dst = pl.ds(obase + ch * CHUNK, CHUNK)
            pltpu.make_async_copy(rs, gs_hbm.at[dst], ss).wait()
            pltpu.make_async_copy(rr, gr_hbm.at[dst], sr).wait()

        issue_g(0, rows_s0, rows_r0, g_s0, g_r0)

        def pair_body(k, _):
            a = 2 * k

            @pl.when(k > 0)
            def _():
                wait_s(a - 1, rows_s1, rows_r1, t_s1, t_r1)

            issue_g(a + 1, rows_s1, rows_r1, g_s1, g_r1)
            wait_g(a, rows_s0, rows_r0, g_s0, g_r0)
            issue_s(a, rows_s0, rows_r0, t_s0, t_r0)
            wait_g(a + 1, rows_s1, rows_r1, g_s1, g_r1)
            issue_s(a + 1, rows_s1, rows_r1, t_s1, t_r1)
            wait_s(a, rows_s0, rows_r0, t_s0, t_r0)
            issue_g(a + 2, rows_s0, rows_r0, g_s0, g_r0)
            return 0

        lax.fori_loop(0, NCH // 2, pair_body, 0)
        last = NCH - 1
        wait_s(last - 1, rows_s1, rows_r1, t_s1, t_r1)
        wait_g(last, rows_s0, rows_r0, g_s0, g_r0)
        issue_s(last, rows_s0, rows_r0, t_s0, t_r0)
        wait_s(last, rows_s0, rows_r0, t_s0, t_r0)

    return pl.kernel(
        body,
        out_type=[jax.ShapeDtypeStruct((EH, H), jnp.float32),
                  jax.ShapeDtypeStruct((EH, H), jnp.float32)],
        mesh=_sc_mesh(),
        scratch_types=[
            pltpu.VMEM((EPW,), jnp.int32),
            pltpu.VMEM((EPW,), jnp.int32),
            pltpu.VMEM((CHUNK, H), jnp.float32),
            pltpu.VMEM((CHUNK, H), jnp.float32),
            pltpu.VMEM((CHUNK, H), jnp.float32),
            pltpu.VMEM((CHUNK, H), jnp.float32),
            pltpu.SemaphoreType.DMA,
            pltpu.SemaphoreType.DMA,
            pltpu.SemaphoreType.DMA,
            pltpu.SemaphoreType.DMA,
            pltpu.SemaphoreType.DMA,
            pltpu.SemaphoreType.DMA,
            pltpu.SemaphoreType.DMA,
            pltpu.SemaphoreType.DMA,
        ],
    )(xs, xr, send, recv)


def _sc_count(recv):
    import jax.experimental.pallas.tpu_sc as plsc

    def body(recv3_hbm, out_hbm, ones_buf, ibuf, acc):
        c = lax.axis_index("c")
        s = lax.axis_index("s")
        wid = c * NS + s

        rows_per_tile = N2 // NS

        def fill_body(val, k, _):
            i = k // (H // 16)
            j = (k % (H // 16)) * 16
            ones_buf[i, pl.ds(j, 16)] = jnp.full((16,), val, jnp.float32)
            return 0

        lax.fori_loop(0, CCHUNK * (H // 16),
                      functools.partial(fill_body, 0.0), 0, unroll=8)
        for t in range(rows_per_tile // CCHUNK):
            pltpu.sync_copy(ones_buf,
                            acc.at[pl.ds(s * rows_per_tile + t * CCHUNK,
                                         CCHUNK)])
        lax.fori_loop(0, CCHUNK * (H // 16),
                      functools.partial(fill_body, 1.0), 0, unroll=8)
        pltpu.sync_copy(recv3_hbm.at[wid], ibuf)
        plsc.subcore_barrier()

        def chunk_body(ch, _):
            pltpu.sync_copy(ones_buf, acc.at[ibuf.at[ch]], add=True)
            return 0

        lax.fori_loop(0, CNCH, chunk_body, 0)
        plsc.subcore_barrier()
        pltpu.sync_copy(acc.at[pl.ds(s * rows_per_tile, rows_per_tile)],
                        out_hbm.at[c, pl.ds(s * rows_per_tile, rows_per_tile)])

    p = pl.kernel(
        body,
        out_type=jax.ShapeDtypeStruct((NC, N2, H), jnp.float32),
        mesh=_sc_mesh(),
        scratch_types=[
            pltpu.VMEM((CCHUNK, H), jnp.float32),
            pltpu.VMEM((CNCH, CCHUNK), jnp.int32),
            pltpu.VMEM_SHARED((N2, H), jnp.float32),
        ],
    )(recv.reshape(NW, CNCH, CCHUNK))
    return p[0], p[1]


def _sc_scatter_add(m, recv4, part):
    import jax.experimental.pallas.tpu_sc as plsc
    w = m.shape[1]

    def body(m_hbm, recv4_hbm, out_hbm, buf0, buf1, ibuf, acc, sm0, sm1):
        c = lax.axis_index("c")
        s = lax.axis_index("s")
        wid = c * NS + s
        base0 = wid * EPW

        def zero_body(k, _):
            i = k // (w // 16)
            j = (k % (w // 16)) * 16
            buf0[i, pl.ds(j, 16)] = jnp.zeros((16,), jnp.float32)
            return 0

        lax.fori_loop(0, CHUNK * (w // 16), zero_body, 0, unroll=8)
        rows_per_tile = N2 // NS
        for t in range(rows_per_tile // CHUNK):
            pltpu.sync_copy(buf0, acc.at[pl.ds(s * rows_per_tile + t * CHUNK,
                                               CHUNK)])
        pltpu.sync_copy(recv4_hbm.at[part, wid], ibuf)
        plsc.subcore_barrier()

        def issue(ch, buf, sem):
            pltpu.async_copy(m_hbm.at[pl.ds(base0 + ch * CHUNK, CHUNK)],
                             buf, sem)

        def wait(ch, buf, sem):
            pltpu.make_async_copy(
                m_hbm.at[pl.ds(base0 + ch * CHUNK, CHUNK)], buf, sem).wait()

        issue(0, buf0, sm0)

        def pair_body(k, _):
            a = 2 * k
            issue(a + 1, buf1, sm1)
            wait(a, buf0, sm0)
            pltpu.sync_copy(buf0, acc.at[ibuf.at[a]], add=True)
            issue(a + 2, buf0, sm0)
            wait(a + 1, buf1, sm1)
            pltpu.sync_copy(buf1, acc.at[ibuf.at[a + 1]], add=True)
            return 0

        lax.fori_loop(0, NCH // 2, pair_body, 0)
        last = NCH - 1
        wait(last, buf0, sm0)
        pltpu.sync_copy(buf0, acc.at[ibuf.at[last]], add=True)
        plsc.subcore_barrier()
        pltpu.sync_copy(acc.at[pl.ds(s * rows_per_tile, rows_per_tile)],
                        out_hbm.at[c, pl.ds(s * rows_per_tile, rows_per_tile)])

    p = pl.kernel(
        body,
        out_type=jax.ShapeDtypeStruct((NC, N2, w), jnp.float32),
        mesh=_sc_mesh(),
        scratch_types=[
            pltpu.VMEM((CHUNK, w), jnp.float32),
            pltpu.VMEM((CHUNK, w), jnp.float32),
            pltpu.VMEM((NCH, CHUNK), jnp.int32),
            pltpu.VMEM_SHARED((N2, w), jnp.float32),
            pltpu.SemaphoreType.DMA,
            pltpu.SemaphoreType.DMA,
        ],
    )(m, recv4)
    return p[0], p[1]


def kernel(x, edge_attr, edges,
           l0_m0_w, l0_m0_b, l0_m1_w, l0_m1_b, l0_u0_w, l0_u0_b, l0_u1_w, l0_u1_b,
           l1_m0_w, l1_m0_b, l1_m1_w, l1_m1_b, l1_u0_w, l1_u0_b, l1_u1_w, l1_u1_b,
           l2_m0_w, l2_m0_b, l2_m1_w, l2_m1_b, l2_u0_w, l2_u0_b, l2_u1_w, l2_u1_b,
           l3_m0_w, l3_m0_b, l3_m1_w, l3_m1_b, l3_u0_w, l3_u0_b, l3_u1_w, l3_u1_b,
           o0_w, o0_b, o1_w, o1_b, o2_w, o2_b):
    send = edges[0]
    recv = edges[1]
    recv4 = recv.reshape(PARTS, NW, NCH, CHUNK)

    m0w = [l0_m0_w, l1_m0_w, l2_m0_w, l3_m0_w]
    m0b = [l0_m0_b, l1_m0_b, l2_m0_b, l3_m0_b]
    m1w = [l0_m1_w, l1_m1_w, l2_m1_w, l3_m1_w]
    m1b = [l0_m1_b, l1_m1_b, l2_m1_b, l3_m1_b]
    u0w = [l0_u0_w, l1_u0_w, l2_u0_w, l3_u0_w]
    u0b = [l0_u0_b, l1_u0_b, l2_u0_b, l3_u0_b]
    u1w = [l0_u1_w, l1_u1_w, l2_u1_w, l3_u1_w]
    u1b = [l0_u1_b, l1_u1_b, l2_u1_b, l3_u1_b]

    ws = [None] + [w[:H] for w in m0w[1:]]
    wr = [None] + [w[H:2 * H] for w in m0w[1:]]
    we = [None] + [w[2 * H:] for w in m0w[1:]]

    o2w_pad = jnp.zeros((H, H), jnp.float32).at[:, :3].set(o2_w)
    o2b_pad = jnp.zeros((H,), jnp.float32).at[:3].set(o2_b)

    c0, c1 = _sc_count(recv)
    mh = [_edge0(edge_attr, m0w[0], m0b[0], m1w[0], m1b[0], h)
          for h in range(PARTS)]
    ps = []
    for h in range(PARTS):
        ps += list(_sc_scatter_add(mh[h], recv4, h))
    x, xs, xr, inv = _node0(x, ps, c0, c1,
                            u0w[0], u0b[0], u1w[0], u1b[0],
                            ws[1], wr[1], m0b[1])
    ea = mh

    for i in (1, 2, 3):
        g = [_sc_gather_add(xs, xr, send, recv, h) for h in range(PARTS)]
        m = [_edge(ea[h], g[h][0], g[h][1], we[i], m1w[i], m1b[i])
             for h in range(PARTS)]
        ps = []
        for h in range(PARTS):
            ps += list(_sc_scatter_add(m[h], recv4, h))
        if i < 3:
            x, xs, xr = _node(x, ps, inv, u0w[i], u0b[i], u1w[i], u1b[i],
                              ws[i + 1], wr[i + 1], m0b[i + 1])
            ea = m
        else:
            out = _node_final(x, ps, inv, u0w[i], u0b[i], u1w[i], u1b[i],
                              o0_w, o0_b, o1_w, o1_b, o2w_pad, o2b_pad)
    return out[:, :3]


# speedup vs baseline: 1.0203x; 1.0203x over previous
import functools

import jax
import jax.numpy as jnp
from jax import lax
from jax.experimental import pallas as pl
from jax.experimental.pallas import tpu as pltpu

N = 10000
E = 320000
D = 128
H = 128
NEF = 144

BE = 4000
BN = 1000

NC = 2
NS = 16
NW = NC * NS
N2 = 10240

PARTS = 2
EH = E // PARTS
EPW = EH // NW
CHUNK = 40
NCH = EPW // CHUNK

CCHUNK = 80
CEPW = E // NW
CNCH = CEPW // CCHUNK


def _silu(v):
    return v * jax.nn.sigmoid(v)


def _edge0_body(ea_ref, w0_ref, b0_ref, w1_ref, b1_ref, out_ref):
    pre = jnp.dot(ea_ref[...], w0_ref[...], preferred_element_type=jnp.float32)
    h = _silu(pre + b0_ref[...])
    m = _silu(jnp.dot(h, w1_ref[...], preferred_element_type=jnp.float32)
              + b1_ref[...])
    out_ref[...] = m


def _edge0(ea, w0, b0, w1, b1, part):
    nb = EH // BE
    return pl.pallas_call(
        _edge0_body,
        grid=(nb,),
        in_specs=[
            pl.BlockSpec((BE, NEF), lambda i: (i + part * nb, 0)),
            pl.BlockSpec((NEF, H), lambda i: (0, 0)),
            pl.BlockSpec((1, H), lambda i: (0, 0)),
            pl.BlockSpec((H, H), lambda i: (0, 0)),
            pl.BlockSpec((1, H), lambda i: (0, 0)),
        ],
        out_specs=pl.BlockSpec((BE, H), lambda i: (i, 0)),
        out_shape=jax.ShapeDtypeStruct((EH, H), jnp.float32),
    )(ea, w0, b0.reshape(1, H), w1, b1.reshape(1, H))


def _edge_body(ea_ref, gs_ref, gr_ref, we_ref, w1_ref, b1_ref, out_ref):
    pre = jnp.dot(ea_ref[...], we_ref[...], preferred_element_type=jnp.float32)
    h = _silu(pre + (gs_ref[...] + gr_ref[...]))
    m = _silu(jnp.dot(h, w1_ref[...], preferred_element_type=jnp.float32)
              + b1_ref[...])
    out_ref[...] = m


def _edge(ea, gs, gr, we, w1, b1):
    nb = EH // BE
    return pl.pallas_call(
        _edge_body,
        grid=(nb,),
        in_specs=[
            pl.BlockSpec((BE, H), lambda i: (i, 0)),
            pl.BlockSpec((BE, H), lambda i: (i, 0)),
            pl.BlockSpec((BE, H), lambda i: (i, 0)),
            pl.BlockSpec((H, H), lambda i: (0, 0)),
            pl.BlockSpec((H, H), lambda i: (0, 0)),
            pl.BlockSpec((1, H), lambda i: (0, 0)),
        ],
        out_specs=pl.BlockSpec((BE, H), lambda i: (i, 0)),
        out_shape=jax.ShapeDtypeStruct((EH, H), jnp.float32),
    )(ea, gs, gr, we, w1, b1.reshape(1, H))


def _node0_body(x_ref, pa_ref, pb_ref, cc_ref,
                u0w_ref, u0b_ref, u1w_ref, u1b_ref,
                wsn_ref, wrn_ref, b0n_ref,
                x_out, xs_out, xr_out, inv_out):
    s = (pa_ref[0] + pa_ref[1]) + (pb_ref[0] + pb_ref[1])
    c = cc_ref[0, :, :1] + cc_ref[1, :, :1]
    inv = 1.0 / jnp.maximum(c, 1.0)
    x1 = x_ref[...] + s * inv
    u = _silu(jnp.dot(x1, u0w_ref[...], preferred_element_type=jnp.float32)
              + u0b_ref[...])
    x2 = x1 + jnp.dot(u, u1w_ref[...], preferred_element_type=jnp.float32) \
        + u1b_ref[...]
    x_out[...] = x2
    xs_out[...] = jnp.dot(x2, wsn_ref[...],
                          preferred_element_type=jnp.float32) + b0n_ref[...]
    xr_out[...] = jnp.dot(x2, wrn_ref[...],
                          preferred_element_type=jnp.float32)
    inv_out[...] = inv


def _node0(x, pa, pb, cc, u0w, u0b, u1w, u1b, wsn, wrn, b0n):
    pspec = pl.BlockSpec((NC, BN, H), lambda i: (0, i, 0))
    return pl.pallas_call(
        _node0_body,
        grid=(N // BN,),
        in_specs=[pl.BlockSpec((BN, H), lambda i: (i, 0))] + [pspec] * 3 + [
            pl.BlockSpec((H, 2 * H), lambda i: (0, 0)),
            pl.BlockSpec((1, 2 * H), lambda i: (0, 0)),
            pl.BlockSpec((2 * H, H), lambda i: (0, 0)),
            pl.BlockSpec((1, H), lambda i: (0, 0)),
            pl.BlockSpec((H, H), lambda i: (0, 0)),
            pl.BlockSpec((H, H), lambda i: (0, 0)),
            pl.BlockSpec((1, H), lambda i: (0, 0)),
        ],
        out_specs=[
            pl.BlockSpec((BN, H), lambda i: (i, 0)),
            pl.BlockSpec((BN, H), lambda i: (i, 0)),
            pl.BlockSpec((BN, H), lambda i: (i, 0)),
            pl.BlockSpec((BN, 1), lambda i: (i, 0)),
        ],
        out_shape=[
            jax.ShapeDtypeStruct((N, H), jnp.float32),
            jax.ShapeDtypeStruct((N, H), jnp.float32),
            jax.ShapeDtypeStruct((N, H), jnp.float32),
            jax.ShapeDtypeStruct((N, 1), jnp.float32),
        ],
    )(x, pa, pb, cc,
      u0w, u0b.reshape(1, 2 * H), u1w, u1b.reshape(1, H),
      wsn, wrn, b0n.reshape(1, H))


def _node_body(x_ref, pa_ref, pb_ref, inv_ref,
               u0w_ref, u0b_ref, u1w_ref, u1b_ref,
               wsn_ref, wrn_ref, b0n_ref,
               x_out, xs_out, xr_out):
    s = (pa_ref[0] + pa_ref[1]) + (pb_ref[0] + pb_ref[1])
    x1 = x_ref[...] + s * inv_ref[...]
    u = _silu(jnp.dot(x1, u0w_ref[...], preferred_element_type=jnp.float32)
              + u0b_ref[...])
    x2 = x1 + jnp.dot(u, u1w_ref[...], preferred_element_type=jnp.float32) \
        + u1b_ref[...]
    x_out[...] = x2
    xs_out[...] = jnp.dot(x2, wsn_ref[...],
                          preferred_element_type=jnp.float32) + b0n_ref[...]
    xr_out[...] = jnp.dot(x2, wrn_ref[...],
                          preferred_element_type=jnp.float32)


def _node(x, pa, pb, inv, u0w, u0b, u1w, u1b, wsn, wrn, b0n):
    pspec = pl.BlockSpec((NC, BN, H), lambda i: (0, i, 0))
    return pl.pallas_call(
        _node_body,
        grid=(N // BN,),
        in_specs=[pl.BlockSpec((BN, H), lambda i: (i, 0))] + [pspec] * 2 + [
            pl.BlockSpec((BN, 1), lambda i: (i, 0)),
            pl.BlockSpec((H, 2 * H), lambda i: (0, 0)),
            pl.BlockSpec((1, 2 * H), lambda i: (0, 0)),
            pl.BlockSpec((2 * H, H), lambda i: (0, 0)),
            pl.BlockSpec((1, H), lambda i: (0, 0)),
            pl.BlockSpec((H, H), lambda i: (0, 0)),
            pl.BlockSpec((H, H), lambda i: (0, 0)),
            pl.BlockSpec((1, H), lambda i: (0, 0)),
        ],
        out_specs=[
            pl.BlockSpec((BN, H), lambda i: (i, 0)),
            pl.BlockSpec((BN, H), lambda i: (i, 0)),
            pl.BlockSpec((BN, H), lambda i: (i, 0)),
        ],
        out_shape=[
            jax.ShapeDtypeStruct((N, H), jnp.float32),
            jax.ShapeDtypeStruct((N, H), jnp.float32),
            jax.ShapeDtypeStruct((N, H), jnp.float32),
        ],
    )(x, pa, pb, inv,
      u0w, u0b.reshape(1, 2 * H), u1w, u1b.reshape(1, H),
      wsn, wrn, b0n.reshape(1, H))


def _node_final_body(x_ref, pa_ref, pb_ref, inv_ref,
                     u0w_ref, u0b_ref, u1w_ref, u1b_ref,
                     o0w_ref, o0b_ref, o1w_ref, o1b_ref,
                     o2w_ref, o2b_ref, out_ref):
    s = (pa_ref[0] + pa_ref[1]) + (pb_ref[0] + pb_ref[1])
    x1 = x_ref[...] + s * inv_ref[...]
    u = _silu(jnp.dot(x1, u0w_ref[...], preferred_element_type=jnp.float32)
              + u0b_ref[...])
    x2 = x1 + jnp.dot(u, u1w_ref[...], preferred_element_type=jnp.float32) \
        + u1b_ref[...]
    h = _silu(jnp.dot(x2, o0w_ref[...], preferred_element_type=jnp.float32)
              + o0b_ref[...])
    h = _silu(jnp.dot(h, o1w_ref[...], preferred_element_type=jnp.float32)
              + o1b_ref[...])
    out_ref[...] = jnp.dot(h, o2w_ref[...], preferred_element_type=jnp.float32) \
        + o2b_ref[...]


def _node_final(x, pa, pb, inv, u0w, u0b, u1w, u1b,
                o0w, o0b, o1w, o1b, o2w_pad, o2b_pad):
    pspec = pl.BlockSpec((NC, BN, H), lambda i: (0, i, 0))
    wspec = pl.BlockSpec((H, H), lambda i: (0, 0))
    bspec = pl.BlockSpec((1, H), lambda i: (0, 0))
    return pl.pallas_call(
        _node_final_body,
        grid=(N // BN,),
        in_specs=[pl.BlockSpec((BN, H), lambda i: (i, 0))] + [pspec] * 2 + [
            pl.BlockSpec((BN, 1), lambda i: (i, 0)),
            pl.BlockSpec((H, 2 * H), lambda i: (0, 0)),
            pl.BlockSpec((1, 2 * H), lambda i: (0, 0)),
            pl.BlockSpec((2 * H, H), lambda i: (0, 0)),
            bspec, wspec, bspec, wspec, bspec, wspec, bspec,
        ],
        out_specs=pl.BlockSpec((BN, H), lambda i: (i, 0)),
        out_shape=jax.ShapeDtypeStruct((N, H), jnp.float32),
    )(x, pa, pb, inv,
      u0w, u0b.reshape(1, 2 * H), u1w, u1b.reshape(1, H),
      o0w, o0b.reshape(1, H), o1w, o1b.reshape(1, H), o2w_pad,
      o2b_pad.reshape(1, H))


def _sc_mesh():
    import jax.experimental.pallas.tpu_sc as plsc
    return plsc.VectorSubcoreMesh(core_axis_name="c", subcore_axis_name="s")


def _sc_gather_add(xs, xr, send, recv, part):
    def body(xs_hbm, xr_hbm, send_hbm, recv_hbm, gs_hbm, gr_hbm,
             sidx, ridx, rows_s0, rows_r0, rows_s1, rows_r1,
             g_s0, g_r0, g_s1, g_r1, t_s0, t_r0, t_s1, t_r1):
        wid = lax.axis_index("c") * NS + lax.axis_index("s")
        ebase = part * EH + wid * EPW
        obase = wid * EPW
        pltpu.sync_copy(send_hbm.at[pl.ds(ebase, EPW)], sidx)
        pltpu.sync_copy(recv_hbm.at[pl.ds(ebase, EPW)], ridx)

        def issue_g(ch, rs, rr, ss, sr):
            pltpu.async_copy(xs_hbm.at[sidx.at[pl.ds(ch * CHUNK, CHUNK)]],
                             rs, ss)
            pltpu.async_copy(xr_hbm.at[ridx.at[pl.ds(ch * CHUNK, CHUNK)]],
                             rr, sr)

        def wait_g(ch, rs, rr, ss, sr):
            pltpu.make_async_copy(
                xs_hbm.at[sidx.at[pl.ds(ch * CHUNK, CHUNK)]], rs, ss).wait()
            pltpu.make_async_copy(
                xr_hbm.at[ridx.at[pl.ds(ch * CHUNK, CHUNK)]], rr, sr).wait()

        def issue_s(ch, rs, rr, ss, sr):
            dst = pl.ds(obase + ch * CHUNK, CHUNK)
            pltpu.async_copy(rs, gs_hbm.at[dst], ss)
            pltpu.async_copy(rr, gr_hbm.at[dst], sr)

        def wait_s(ch, rs, rr, ss, sr):
            dst = pl.ds(obase + ch * CHUNK, CHUNK)
            pltpu.make_async_copy(rs, gs_hbm.at[dst], ss).wait()
            pltpu.make_async_copy(rr, gr_hbm.at[dst], sr).wait()

        issue_g(0, rows_s0, rows_r0, g_s0, g_r0)

        def pair_body(k, _):
            a = 2 * k

            @pl.when(k > 0)
            def _():
                wait_s(a - 1, rows_s1, rows_r1, t_s1, t_r1)

            issue_g(a + 1, rows_s1, rows_r1, g_s1, g_r1)
            wait_g(a, rows_s0, rows_r0, g_s0, g_r0)
            issue_s(a, rows_s0, rows_r0, t_s0, t_r0)
            wait_g(a + 1, rows_s1, rows_r1, g_s1, g_r1)
            issue_s(a + 1, rows_s1, rows_r1, t_s1, t_r1)
            wait_s(a, rows_s0, rows_r0, t_s0, t_r0)
            issue_g(a + 2, rows_s0, rows_r0, g_s0, g_r0)
            return 0

        lax.fori_loop(0, NCH // 2, pair_body, 0)
        last = NCH - 1
        wait_s(last - 1, rows_s1, rows_r1, t_s1, t_r1)
        wait_g(last, rows_s0, rows_r0, g_s0, g_r0)
        issue_s(last, rows_s0, rows_r0, t_s0, t_r0)
        wait_s(last, rows_s0, rows_r0, t_s0, t_r0)

    return pl.kernel(
        body,
        out_type=[jax.ShapeDtypeStruct((EH, H), jnp.float32),
                  jax.ShapeDtypeStruct((EH, H), jnp.float32)],
        mesh=_sc_mesh(),
        scratch_types=[
            pltpu.VMEM((EPW,), jnp.int32),
            pltpu.VMEM((EPW,), jnp.int32),
            pltpu.VMEM((CHUNK, H), jnp.float32),
            pltpu.VMEM((CHUNK, H), jnp.float32),
            pltpu.VMEM((CHUNK, H), jnp.float32),
            pltpu.VMEM((CHUNK, H), jnp.float32),
            pltpu.SemaphoreType.DMA,
            pltpu.SemaphoreType.DMA,
            pltpu.SemaphoreType.DMA,
            pltpu.SemaphoreType.DMA,
            pltpu.SemaphoreType.DMA,
            pltpu.SemaphoreType.DMA,
            pltpu.SemaphoreType.DMA,
            pltpu.SemaphoreType.DMA,
        ],
    )(xs, xr, send, recv)


def _sc_count(recv):
    import jax.experimental.pallas.tpu_sc as plsc

    def body(recv3_hbm, out_hbm, ones_buf, ibuf, acc):
        c = lax.axis_index("c")
        s = lax.axis_index("s")
        wid = c * NS + s

        rows_per_tile = N2 // NS

        def fill_body(val, k, _):
            i = k // (H // 16)
            j = (k % (H // 16)) * 16
            ones_buf[i, pl.ds(j, 16)] = jnp.full((16,), val, jnp.float32)
            return 0

        lax.fori_loop(0, CCHUNK * (H // 16),
                      functools.partial(fill_body, 0.0), 0, unroll=8)
        for t in range(rows_per_tile // CCHUNK):
            pltpu.sync_copy(ones_buf,
                            acc.at[pl.ds(s * rows_per_tile + t * CCHUNK,
                                         CCHUNK)])
        lax.fori_loop(0, CCHUNK * (H // 16),
                      functools.partial(fill_body, 1.0), 0, unroll=8)
        pltpu.sync_copy(recv3_hbm.at[wid], ibuf)
        plsc.subcore_barrier()

        def chunk_body(ch, _):
            pltpu.sync_copy(ones_buf, acc.at[ibuf.at[ch]], add=True)
            return 0

        lax.fori_loop(0, CNCH, chunk_body, 0)
        plsc.subcore_barrier()
        pltpu.sync_copy(acc.at[pl.ds(s * rows_per_tile, rows_per_tile)],
                        out_hbm.at[c, pl.ds(s * rows_per_tile, rows_per_tile)])

    p = pl.kernel(
        body,
        out_type=jax.ShapeDtypeStruct((NC, N2, H), jnp.float32),
        mesh=_sc_mesh(),
        scratch_types=[
            pltpu.VMEM((CCHUNK, H), jnp.float32),
            pltpu.VMEM((CNCH, CCHUNK), jnp.int32),
            pltpu.VMEM_SHARED((N2, H), jnp.float32),
        ],
    )(recv.reshape(NW, CNCH, CCHUNK))
    return p


def _sc_scatter_add(m, recv4, part):
    import jax.experimental.pallas.tpu_sc as plsc
    w = m.shape[1]

    def body(m_hbm, recv4_hbm, out_hbm, buf0, buf1, ibuf, acc, sm0, sm1):
        c = lax.axis_index("c")
        s = lax.axis_index("s")
        wid = c * NS + s
        base0 = wid * EPW

        def zero_body(k, _):
            i = k // (w // 16)
            j = (k % (w // 16)) * 16
            buf0[i, pl.ds(j, 16)] = jnp.zeros((16,), jnp.float32)
            return 0

        lax.fori_loop(0, CHUNK * (w // 16), zero_body, 0, unroll=8)
        rows_per_tile = N2 // NS
        for t in range(rows_per_tile // CHUNK):
            pltpu.sync_copy(buf0, acc.at[pl.ds(s * rows_per_tile + t * CHUNK,
                                               CHUNK)])
        pltpu.sync_copy(recv4_hbm.at[part, wid], ibuf)
        plsc.subcore_barrier()

        def issue(ch, buf, sem):
            pltpu.async_copy(m_hbm.at[pl.ds(base0 + ch * CHUNK, CHUNK)],
                             buf, sem)

        def wait(ch, buf, sem):
            pltpu.make_async_copy(
                m_hbm.at[pl.ds(base0 + ch * CHUNK, CHUNK)], buf, sem).wait()

        issue(0, buf0, sm0)

        def pair_body(k, _):
            a = 2 * k
            issue(a + 1, buf1, sm1)
            wait(a, buf0, sm0)
            pltpu.sync_copy(buf0, acc.at[ibuf.at[a]], add=True)
            issue(a + 2, buf0, sm0)
            wait(a + 1, buf1, sm1)
            pltpu.sync_copy(buf1, acc.at[ibuf.at[a + 1]], add=True)
            return 0

        lax.fori_loop(0, NCH // 2, pair_body, 0)
        last = NCH - 1
        wait(last, buf0, sm0)
        pltpu.sync_copy(buf0, acc.at[ibuf.at[last]], add=True)
        plsc.subcore_barrier()
        pltpu.sync_copy(acc.at[pl.ds(s * rows_per_tile, rows_per_tile)],
                        out_hbm.at[c, pl.ds(s * rows_per_tile, rows_per_tile)])

    p = pl.kernel(
        body,
        out_type=jax.ShapeDtypeStruct((NC, N2, w), jnp.float32),
        mesh=_sc_mesh(),
        scratch_types=[
            pltpu.VMEM((CHUNK, w), jnp.float32),
            pltpu.VMEM((CHUNK, w), jnp.float32),
            pltpu.VMEM((NCH, CHUNK), jnp.int32),
            pltpu.VMEM_SHARED((N2, w), jnp.float32),
            pltpu.SemaphoreType.DMA,
            pltpu.SemaphoreType.DMA,
        ],
    )(m, recv4)
    return p


def kernel(x, edge_attr, edges,
           l0_m0_w, l0_m0_b, l0_m1_w, l0_m1_b, l0_u0_w, l0_u0_b, l0_u1_w, l0_u1_b,
           l1_m0_w, l1_m0_b, l1_m1_w, l1_m1_b, l1_u0_w, l1_u0_b, l1_u1_w, l1_u1_b,
           l2_m0_w, l2_m0_b, l2_m1_w, l2_m1_b, l2_u0_w, l2_u0_b, l2_u1_w, l2_u1_b,
           l3_m0_w, l3_m0_b, l3_m1_w, l3_m1_b, l3_u0_w, l3_u0_b, l3_u1_w, l3_u1_b,
           o0_w, o0_b, o1_w, o1_b, o2_w, o2_b):
    send = edges[0]
    recv = edges[1]
    recv4 = recv.reshape(PARTS, NW, NCH, CHUNK)

    m0w = [l0_m0_w, l1_m0_w, l2_m0_w, l3_m0_w]
    m0b = [l0_m0_b, l1_m0_b, l2_m0_b, l3_m0_b]
    m1w = [l0_m1_w, l1_m1_w, l2_m1_w, l3_m1_w]
    m1b = [l0_m1_b, l1_m1_b, l2_m1_b, l3_m1_b]
    u0w = [l0_u0_w, l1_u0_w, l2_u0_w, l3_u0_w]
    u0b = [l0_u0_b, l1_u0_b, l2_u0_b, l3_u0_b]
    u1w = [l0_u1_w, l1_u1_w, l2_u1_w, l3_u1_w]
    u1b = [l0_u1_b, l1_u1_b, l2_u1_b, l3_u1_b]

    ws = [None] + [w[:H] for w in m0w[1:]]
    wr = [None] + [w[H:2 * H] for w in m0w[1:]]
    we = [None] + [w[2 * H:] for w in m0w[1:]]

    o2w_pad = jnp.zeros((H, H), jnp.float32).at[:, :3].set(o2_w)
    o2b_pad = jnp.zeros((H,), jnp.float32).at[:3].set(o2_b)

    cc = _sc_count(recv)
    mh = [_edge0(edge_attr, m0w[0], m0b[0], m1w[0], m1b[0], h)
          for h in range(PARTS)]
    pa, pb = [_sc_scatter_add(mh[h], recv4, h) for h in range(PARTS)]
    x, xs, xr, inv = _node0(x, pa, pb, cc,
                            u0w[0], u0b[0], u1w[0], u1b[0],
                            ws[1], wr[1], m0b[1])
    ea = mh

    for i in (1, 2, 3):
        g = [_sc_gather_add(xs, xr, send, recv, h) for h in range(PARTS)]
        m = [_edge(ea[h], g[h][0], g[h][1], we[i], m1w[i], m1b[i])
             for h in range(PARTS)]
        pa, pb = [_sc_scatter_add(m[h], recv4, h) for h in range(PARTS)]
        if i < 3:
            x, xs, xr = _node(x, pa, pb, inv, u0w[i], u0b[i], u1w[i], u1b[i],
                              ws[i + 1], wr[i + 1], m0b[i + 1])
            ea = m
        else:
            out = _node_final(x, pa, pb, inv,
                              u0w[i], u0b[i], u1w[i], u1b[i],
                              o0_w, o0_b, o1_w, o1_b, o2w_pad, o2b_pad)
    return out[:, :3]
